# Initial kernel scaffold; baseline (speedup 1.0000x reference)
import jax
import jax.numpy as jnp
from jax.experimental import pallas as pl


def kernel(hidden_states, router_w, expert_bias, gate_w, down_w, shared_gate_w, shared_down_w):
    raise NotImplementedError("write your pallas kernel here")


# speedup vs baseline: 1.7542x; 1.7542x over previous
import functools

import jax
import jax.numpy as jnp
from jax.experimental import pallas as pl
from jax.experimental.pallas import tpu as pltpu

TOPK = 2
SCALE = 1.0


def _moe_body(x_ref, rw_ref, b_ref, gw_ref, dw_ref, sgw_ref, sdw_ref, o_ref,
              *, n_experts, tb):
    x = x_ref[...]
    logits = jax.lax.dot_general(
        x, rw_ref[...], (((1,), (1,)), ((), ())),
        preferred_element_type=jnp.float32) + b_ref[...]
    probs = jax.nn.sigmoid(logits)
    iota = jax.lax.broadcasted_iota(jnp.int32, (tb, n_experts), 1)
    m1 = jnp.max(probs, axis=1, keepdims=True)
    idx1 = jnp.min(jnp.where(probs >= m1, iota, n_experts), axis=1,
                   keepdims=True)
    sel1 = iota == idx1
    probs2 = jnp.where(sel1, -jnp.inf, probs)
    m2 = jnp.max(probs2, axis=1, keepdims=True)
    idx2 = jnp.min(jnp.where(probs2 >= m2, iota, n_experts), axis=1,
                   keepdims=True)
    sel2 = iota == idx2
    denom = m1 + m2 + 1e-9
    combine = (jnp.where(sel1, m1, 0.0) + jnp.where(sel2, m2, 0.0)) \
        / denom * SCALE

    xb = x.astype(jnp.bfloat16)

    def ffn(xin, gw, dw):
        h = jax.lax.dot_general(xin, gw, (((1,), (1,)), ((), ())),
                                preferred_element_type=jnp.float32)
        a = h * jax.nn.sigmoid(h)
        return jax.lax.dot_general(a.astype(jnp.bfloat16), dw,
                                   (((1,), (1,)), ((), ())),
                                   preferred_element_type=jnp.float32)

    acc = ffn(xb, sgw_ref[...], sdw_ref[...])
    for e in range(n_experts):
        pe = ffn(xb, gw_ref[e], dw_ref[e])
        acc = acc + combine[:, e:e + 1] * pe
    o_ref[...] = acc


def kernel(hidden_states, router_w, expert_bias, gate_w, down_w,
           shared_gate_w, shared_down_w):
    b, s, h = hidden_states.shape
    t = b * s
    e, f, _ = gate_w.shape
    x = hidden_states.reshape(t, h)
    bias2d = expert_bias.reshape(1, e)
    gate_bf = gate_w.astype(jnp.bfloat16)
    down_bf = down_w.astype(jnp.bfloat16)
    sgw_bf = shared_gate_w.astype(jnp.bfloat16)
    sdw_bf = shared_down_w.astype(jnp.bfloat16)

    tb = min(512, t)
    grid = (t // tb,)
    body = functools.partial(_moe_body, n_experts=e, tb=tb)
    out = pl.pallas_call(
        body,
        grid=grid,
        in_specs=[
            pl.BlockSpec((tb, h), lambda i: (i, 0)),
            pl.BlockSpec((e, h), lambda i: (0, 0)),
            pl.BlockSpec((1, e), lambda i: (0, 0)),
            pl.BlockSpec((e, f, h), lambda i: (0, 0, 0)),
            pl.BlockSpec((e, h, f), lambda i: (0, 0, 0)),
            pl.BlockSpec((f, h), lambda i: (0, 0)),
            pl.BlockSpec((h, f), lambda i: (0, 0)),
        ],
        out_specs=pl.BlockSpec((tb, h), lambda i: (i, 0)),
        out_shape=jax.ShapeDtypeStruct((t, h), jnp.float32),
        compiler_params=pltpu.CompilerParams(
            dimension_semantics=("arbitrary",)),
    )(x, router_w, bias2d, gate_bf, down_bf, sgw_bf, sdw_bf)
    return out.reshape(b, s, h)
